# baseline (speedup 1.0000x reference)
import functools

import numpy as np
import jax
import jax.numpy as jnp
from jax import lax
from jax.experimental import pallas as pl
from jax.experimental.pallas import tpu as pltpu
from jax.experimental.pallas import tpu_sc as plsc

B = 8192
C = 1024
HDIM = 16
S = 8
R = 128
G = B // R
BS = B * S

TINY = np.float32(np.finfo(np.float32).tiny)
EXP_ONE = 0x3F800000
LP_CONST = np.float32(-np.log(5.0) - 0.5 * np.log(2.0 * np.pi))

KS1 = 42
KS2 = 42 ^ 0x1BD11BDA


def _i32(v):
    return jnp.int32(np.int32(np.uint32(v & 0xFFFFFFFF).astype(np.int32)))


def _rotl(x, r):
    return lax.shift_left(x, _i32(r)) | lax.shift_right_logical(x, _i32(32 - r))


def _threefry_bits(x1):
    x0 = x1
    x1 = x0 ^ _rotl(x1, 13)
    for r in (15, 26, 6):
        x0 = x0 + x1
        x1 = x0 ^ _rotl(x1, r)
    x0 = x0 + _i32(KS1)
    x1 = x1 + _i32(KS2 + 1)
    for r in (17, 29, 16, 24):
        x0 = x0 + x1
        x1 = x0 ^ _rotl(x1, r)
    x0 = x0 + _i32(KS2)
    x1 = x1 + _i32(2)
    for r in (13, 15, 26, 6):
        x0 = x0 + x1
        x1 = x0 ^ _rotl(x1, r)
    x1 = x1 + _i32(KS1 + 3)
    for r in (17, 29, 16, 24):
        x0 = x0 + x1
        x1 = x0 ^ _rotl(x1, r)
    x0 = x0 + _i32(KS1)
    x1 = x1 + _i32(KS2 + 4)
    for r in (13, 15, 26, 6):
        x0 = x0 + x1
        x1 = x0 ^ _rotl(x1, r)
    x0 = x0 + _i32(KS2)
    x1 = x1 + _i32(5)
    return x0 ^ x1


def _log_softmax_rows(a):
    m = jnp.max(a, axis=-1, keepdims=True)
    sh = a - m
    return sh - jnp.log(jnp.sum(jnp.exp(sh), axis=-1, keepdims=True))


def _tc_body(x_ref, rows_ref, p_ref, w2t_ref, cnt_ref,
             qz_ref, pz_ref, idx_ref, pkl_ref, lpt_ref, lpm_ref, prk_ref):
    g = pl.program_id(0)
    xb = x_ref[...]
    pz_row = rows_ref[0:1, :]
    mu_row = rows_ref[1:2, :]
    opt_row = rows_ref[2:3, :]
    b2_row = rows_ref[3:4, :]
    w1_row = p_ref[0:1, :]
    b1_row = p_ref[1:2, :]

    h = jnp.tanh(xb * w1_row + b1_row)
    qz_raw = lax.dot_general(
        h, w2t_ref[...], (((1,), (0,)), ((), ())),
        precision=lax.Precision.HIGHEST,
        preferred_element_type=jnp.float32) + b2_row
    qz = _log_softmax_rows(qz_raw)
    qz_ref[...] = qz

    pz_ref[...] = jnp.broadcast_to(pz_row, (R, C))

    lp = -0.5 * ((xb - mu_row) / 5.0) ** 2 + LP_CONST
    tos = jnp.sum(opt_row)
    tl = _log_softmax_rows(lp + tos)
    pkl_part = jnp.sum(jnp.exp(qz) * (qz - tl))

    opt_lsm = _log_softmax_rows(opt_row)
    z1 = opt_lsm + lp
    m1 = jnp.max(z1, axis=-1, keepdims=True)
    lpt_part = jnp.sum(m1) + jnp.sum(
        jnp.log(jnp.sum(jnp.exp(z1 - m1), axis=-1)))
    z2 = pz_row + lp
    m2 = jnp.max(z2, axis=-1, keepdims=True)
    lpm_part = jnp.sum(m2) + jnp.sum(
        jnp.log(jnp.sum(jnp.exp(z2 - m2), axis=-1)))

    invB = np.float32(1.0 / B)
    is_last = g == G - 1
    pkl_acc = jnp.where(g == 0, pkl_part, pkl_ref[0, 0] + pkl_part)
    pkl_ref[0, 0] = jnp.where(is_last, pkl_acc * invB, pkl_acc)
    lpt_acc = jnp.where(g == 0, lpt_part, lpt_ref[0, 0] + lpt_part)
    lpt_ref[0, 0] = jnp.where(is_last, lpt_acc * invB, lpt_acc)
    lpm_acc = jnp.where(g == 0, lpm_part, lpm_ref[0, 0] + lpm_part)
    lpm_ref[0, 0] = jnp.where(is_last, lpm_acc * invB, lpm_acc)

    @pl.when(g == 0)
    def _():
        prk_ref[0, 0] = jnp.sum(jnp.exp(pz_row) * (pz_row - opt_lsm))

    x1 = cnt_ref[...] + g * _i32(R * S * C)
    bits = _threefry_bits(x1)
    fb = lax.shift_right_logical(bits, _i32(9)) | _i32(EXP_ONE)
    u = lax.bitcast_convert_type(fb, jnp.float32) - np.float32(1.0)
    pzl_row = rows_ref[4:5, :]
    nscore = jnp.log2(-jnp.log2(u)) - pzl_row
    idx_ref[...] = jnp.argmin(nscore, axis=-1).astype(jnp.int32).reshape(R, S)


def _tc_call(x, rows8, p8, w2t, cnt0):
    scal = jax.ShapeDtypeStruct((1, 1), jnp.float32)
    smem = pl.BlockSpec(memory_space=pltpu.SMEM)
    return pl.pallas_call(
        _tc_body,
        grid=(G,),
        in_specs=[
            pl.BlockSpec((R, 1), lambda g: (g, 0)),
            pl.BlockSpec((8, C), lambda g: (0, 0)),
            pl.BlockSpec((8, HDIM), lambda g: (0, 0)),
            pl.BlockSpec((HDIM, C), lambda g: (0, 0)),
            pl.BlockSpec((R * S, C), lambda g: (0, 0)),
        ],
        out_specs=[
            pl.BlockSpec((R, C), lambda g: (g, 0)),
            pl.BlockSpec((R, C), lambda g: (g, 0)),
            pl.BlockSpec((R, S), lambda g: (g, 0)),
            smem, smem, smem, smem,
        ],
        out_shape=[
            jax.ShapeDtypeStruct((B, C), jnp.float32),
            jax.ShapeDtypeStruct((B, C), jnp.float32),
            jax.ShapeDtypeStruct((B, S), jnp.int32),
            scal, scal, scal, scal,
        ],
    )(x, rows8, p8, w2t, cnt0)


_NC = 2
_NS = 16
_NW = _NC * _NS
_PERW = BS // _NW


def _sc_gather_call(idx_flat, pz_flat, mu_flat):
    mesh = plsc.VectorSubcoreMesh(core_axis_name="c", subcore_axis_name="s")

    @functools.partial(
        pl.kernel,
        mesh=mesh,
        compiler_params=pltpu.CompilerParams(needs_layout_passes=False),
        out_type=[
            jax.ShapeDtypeStruct((BS,), jnp.float32),
            jax.ShapeDtypeStruct((BS,), jnp.float32),
        ],
        scratch_types=[
            pltpu.VMEM((_PERW,), jnp.int32),
            pltpu.VMEM((C,), jnp.float32),
            pltpu.VMEM((C,), jnp.float32),
            pltpu.VMEM((_PERW,), jnp.float32),
            pltpu.VMEM((_PERW,), jnp.float32),
        ],
    )
    def sc_kernel(idx_hbm, pz_hbm, mu_hbm, ls_hbm, loc_hbm,
                  idx_v, pz_v, mu_v, ls_v, loc_v):
        wid = lax.axis_index("s") * _NC + lax.axis_index("c")
        base = wid * _PERW
        pltpu.sync_copy(idx_hbm.at[pl.ds(base, _PERW)], idx_v)
        pltpu.sync_copy(pz_hbm, pz_v)
        pltpu.sync_copy(mu_hbm, mu_v)

        def body(i, carry):
            sl = pl.ds(i * 16, 16)
            iv = idx_v[sl]
            ls_v[sl] = plsc.load_gather(pz_v, [iv])
            loc_v[sl] = plsc.load_gather(mu_v, [iv])
            return carry

        lax.fori_loop(0, _PERW // 16, body, 0)
        pltpu.sync_copy(ls_v, ls_hbm.at[pl.ds(base, _PERW)])
        pltpu.sync_copy(loc_v, loc_hbm.at[pl.ds(base, _PERW)])

    return sc_kernel(idx_flat, pz_flat, mu_flat)


def kernel(x, n_samples, log_theta, w1, b1, w2, b2, log_theta_opt, p_mu):
    pz_row = jax.nn.log_softmax(log_theta, axis=-1)
    mu_row = p_mu.reshape(1, C)
    rows8 = jnp.concatenate(
        [pz_row, mu_row, log_theta_opt, b2.reshape(1, C),
         pz_row * np.float32(1.0 / np.log(2.0)),
         jnp.zeros((3, C), jnp.float32)], axis=0)
    p8 = jnp.concatenate(
        [w1.reshape(1, HDIM), b1.reshape(1, HDIM),
         jnp.zeros((6, HDIM), jnp.float32)], axis=0)
    w2t = w2.T
    cnt0 = (jnp.arange(R * S * C, dtype=jnp.int32).reshape(R * S, C)
            + np.int32(KS1))

    qz, pz, idx, pkl, lpt, lpm, prk = _tc_call(x, rows8, p8, w2t, cnt0)

    idx_flat = idx.reshape(BS)
    ls_flat, loc_flat = _sc_gather_call(idx_flat, pz_row.reshape(C),
                                        p_mu.reshape(C))

    log_s = ls_flat.reshape(B, S)
    px_loc = loc_flat.reshape(B, S, 1)
    px_scale = jnp.full((B, S, 1), np.float32(5.0), jnp.float32)
    return (px_loc, px_scale, idx, log_s, qz, pz,
            prk.reshape(()), pkl.reshape(()),
            lpt.reshape(()), lpm.reshape(()))


# speedup vs baseline: 1.0149x; 1.0149x over previous
import functools

import numpy as np
import jax
import jax.numpy as jnp
from jax import lax
from jax.experimental import pallas as pl
from jax.experimental.pallas import tpu as pltpu
from jax.experimental.pallas import tpu_sc as plsc

B = 8192
C = 1024
HDIM = 16
S = 8
R = 128
G = B // R
BS = B * S

TINY = np.float32(np.finfo(np.float32).tiny)
EXP_ONE = 0x3F800000
LP_CONST = np.float32(-np.log(5.0) - 0.5 * np.log(2.0 * np.pi))

KS1 = 42
KS2 = 42 ^ 0x1BD11BDA


def _i32(v):
    return jnp.int32(np.int32(np.uint32(v & 0xFFFFFFFF).astype(np.int32)))


def _rotl(x, r):
    return lax.shift_left(x, _i32(r)) | lax.shift_right_logical(x, _i32(32 - r))


def _threefry_bits(x1):
    x0 = x1
    x1 = x0 ^ _rotl(x1, 13)
    for r in (15, 26, 6):
        x0 = x0 + x1
        x1 = x0 ^ _rotl(x1, r)
    x0 = x0 + _i32(KS1)
    x1 = x1 + _i32(KS2 + 1)
    for r in (17, 29, 16, 24):
        x0 = x0 + x1
        x1 = x0 ^ _rotl(x1, r)
    x0 = x0 + _i32(KS2)
    x1 = x1 + _i32(2)
    for r in (13, 15, 26, 6):
        x0 = x0 + x1
        x1 = x0 ^ _rotl(x1, r)
    x1 = x1 + _i32(KS1 + 3)
    for r in (17, 29, 16, 24):
        x0 = x0 + x1
        x1 = x0 ^ _rotl(x1, r)
    x0 = x0 + _i32(KS1)
    x1 = x1 + _i32(KS2 + 4)
    for r in (13, 15, 26, 6):
        x0 = x0 + x1
        x1 = x0 ^ _rotl(x1, r)
    x0 = x0 + _i32(KS2)
    x1 = x1 + _i32(5)
    return x0 ^ x1


def _log_softmax_rows(a):
    m = jnp.max(a, axis=-1, keepdims=True)
    sh = a - m
    return sh - jnp.log(jnp.sum(jnp.exp(sh), axis=-1, keepdims=True))


def _tc_body(x_ref, rows_ref, p_ref, w2t_ref, cnt_ref,
             qz_ref, pz_ref, idx_ref, pkl_ref, lpt_ref, lpm_ref, prk_ref):
    g = pl.program_id(0)
    xb = x_ref[...]
    pz_row = rows_ref[0:1, :]
    mu_row = rows_ref[1:2, :]
    opt_row = rows_ref[2:3, :]
    b2_row = rows_ref[3:4, :]
    w1_row = p_ref[0:1, :]
    b1_row = p_ref[1:2, :]

    h = jnp.tanh(xb * w1_row + b1_row)
    qz_raw = lax.dot_general(
        h, w2t_ref[...], (((1,), (0,)), ((), ())),
        precision=lax.Precision.HIGHEST,
        preferred_element_type=jnp.float32) + b2_row
    qz = _log_softmax_rows(qz_raw)
    qz_ref[...] = qz

    pz_ref[...] = jnp.broadcast_to(pz_row, (R, C))

    lp = -0.5 * ((xb - mu_row) / 5.0) ** 2 + LP_CONST
    tos = jnp.sum(opt_row)
    tl = _log_softmax_rows(lp + tos)
    pkl_part = jnp.sum(jnp.exp(qz) * (qz - tl))

    opt_lsm = _log_softmax_rows(opt_row)
    z1 = opt_lsm + lp
    m1 = jnp.max(z1, axis=-1, keepdims=True)
    lpt_part = jnp.sum(m1) + jnp.sum(
        jnp.log(jnp.sum(jnp.exp(z1 - m1), axis=-1)))
    z2 = pz_row + lp
    m2 = jnp.max(z2, axis=-1, keepdims=True)
    lpm_part = jnp.sum(m2) + jnp.sum(
        jnp.log(jnp.sum(jnp.exp(z2 - m2), axis=-1)))

    invB = np.float32(1.0 / B)
    is_last = g == G - 1
    pkl_acc = jnp.where(g == 0, pkl_part, pkl_ref[0, 0] + pkl_part)
    pkl_ref[0, 0] = jnp.where(is_last, pkl_acc * invB, pkl_acc)
    lpt_acc = jnp.where(g == 0, lpt_part, lpt_ref[0, 0] + lpt_part)
    lpt_ref[0, 0] = jnp.where(is_last, lpt_acc * invB, lpt_acc)
    lpm_acc = jnp.where(g == 0, lpm_part, lpm_ref[0, 0] + lpm_part)
    lpm_ref[0, 0] = jnp.where(is_last, lpm_acc * invB, lpm_acc)

    @pl.when(g == 0)
    def _():
        prk_ref[0, 0] = jnp.sum(jnp.exp(pz_row) * (pz_row - opt_lsm))

    x1 = cnt_ref[...] + g * _i32(R * S * C)
    bits = _threefry_bits(x1)
    fb = lax.shift_right_logical(bits, _i32(9)) | _i32(EXP_ONE)
    u = lax.bitcast_convert_type(fb, jnp.float32) - np.float32(1.0)
    nscore = jnp.log(-jnp.log(u)) - pz_row
    idx_ref[...] = jnp.argmin(nscore, axis=-1).astype(jnp.int32).reshape(R, S)


def _tc_call(x, rows8, p8, w2t, cnt0):
    scal = jax.ShapeDtypeStruct((1, 1), jnp.float32)
    smem = pl.BlockSpec(memory_space=pltpu.SMEM)
    return pl.pallas_call(
        _tc_body,
        grid=(G,),
        in_specs=[
            pl.BlockSpec((R, 1), lambda g: (g, 0)),
            pl.BlockSpec((8, C), lambda g: (0, 0)),
            pl.BlockSpec((8, HDIM), lambda g: (0, 0)),
            pl.BlockSpec((HDIM, C), lambda g: (0, 0)),
            pl.BlockSpec((R * S, C), lambda g: (0, 0)),
        ],
        out_specs=[
            pl.BlockSpec((R, C), lambda g: (g, 0)),
            pl.BlockSpec((R, C), lambda g: (g, 0)),
            pl.BlockSpec((R, S), lambda g: (g, 0)),
            smem, smem, smem, smem,
        ],
        out_shape=[
            jax.ShapeDtypeStruct((B, C), jnp.float32),
            jax.ShapeDtypeStruct((B, C), jnp.float32),
            jax.ShapeDtypeStruct((B, S), jnp.int32),
            scal, scal, scal, scal,
        ],
    )(x, rows8, p8, w2t, cnt0)


_NC = 2
_NS = 16
_NW = _NC * _NS
_PERW = BS // _NW


def _sc_gather_call(idx_flat, pz_flat, mu_flat):
    mesh = plsc.VectorSubcoreMesh(core_axis_name="c", subcore_axis_name="s")

    @functools.partial(
        pl.kernel,
        mesh=mesh,
        compiler_params=pltpu.CompilerParams(needs_layout_passes=False),
        out_type=[
            jax.ShapeDtypeStruct((BS,), jnp.float32),
            jax.ShapeDtypeStruct((BS,), jnp.float32),
        ],
        scratch_types=[
            pltpu.VMEM((_PERW,), jnp.int32),
            pltpu.VMEM((C,), jnp.float32),
            pltpu.VMEM((C,), jnp.float32),
            pltpu.VMEM((_PERW,), jnp.float32),
            pltpu.VMEM((_PERW,), jnp.float32),
        ],
    )
    def sc_kernel(idx_hbm, pz_hbm, mu_hbm, ls_hbm, loc_hbm,
                  idx_v, pz_v, mu_v, ls_v, loc_v):
        wid = lax.axis_index("s") * _NC + lax.axis_index("c")
        base = wid * _PERW
        pltpu.sync_copy(idx_hbm.at[pl.ds(base, _PERW)], idx_v)
        pltpu.sync_copy(pz_hbm, pz_v)
        pltpu.sync_copy(mu_hbm, mu_v)

        def body(i, carry):
            sl = pl.ds(i * 16, 16)
            iv = idx_v[sl]
            ls_v[sl] = plsc.load_gather(pz_v, [iv])
            loc_v[sl] = plsc.load_gather(mu_v, [iv])
            return carry

        lax.fori_loop(0, _PERW // 16, body, 0)
        pltpu.sync_copy(ls_v, ls_hbm.at[pl.ds(base, _PERW)])
        pltpu.sync_copy(loc_v, loc_hbm.at[pl.ds(base, _PERW)])

    return sc_kernel(idx_flat, pz_flat, mu_flat)


def kernel(x, n_samples, log_theta, w1, b1, w2, b2, log_theta_opt, p_mu):
    pz_row = jax.nn.log_softmax(log_theta, axis=-1)
    mu_row = p_mu.reshape(1, C)
    rows8 = jnp.concatenate(
        [pz_row, mu_row, log_theta_opt, b2.reshape(1, C),
         jnp.zeros((4, C), jnp.float32)], axis=0)
    p8 = jnp.concatenate(
        [w1.reshape(1, HDIM), b1.reshape(1, HDIM),
         jnp.zeros((6, HDIM), jnp.float32)], axis=0)
    w2t = w2.T
    cnt0 = (jnp.arange(R * S * C, dtype=jnp.int32).reshape(R * S, C)
            + np.int32(KS1))

    qz, pz, idx, pkl, lpt, lpm, prk = _tc_call(x, rows8, p8, w2t, cnt0)

    idx_flat = idx.reshape(BS)
    ls_flat, loc_flat = _sc_gather_call(idx_flat, pz_row.reshape(C),
                                        p_mu.reshape(C))

    log_s = ls_flat.reshape(B, S)
    px_loc = loc_flat.reshape(B, S, 1)
    px_scale = jnp.full((B, S, 1), np.float32(5.0), jnp.float32)
    return (px_loc, px_scale, idx, log_s, qz, pz,
            prk.reshape(()), pkl.reshape(()),
            lpt.reshape(()), lpm.reshape(()))


# speedup vs baseline: 1.0358x; 1.0206x over previous
import functools

import numpy as np
import jax
import jax.numpy as jnp
from jax import lax
from jax.experimental import pallas as pl
from jax.experimental.pallas import tpu as pltpu
from jax.experimental.pallas import tpu_sc as plsc

B = 8192
C = 1024
HDIM = 16
S = 8
R = 128
G = B // R
BS = B * S

TINY = np.float32(np.finfo(np.float32).tiny)
EXP_ONE = 0x3F800000
LP_CONST = np.float32(-np.log(5.0) - 0.5 * np.log(2.0 * np.pi))

KS1 = 42
KS2 = 42 ^ 0x1BD11BDA


def _i32(v):
    return jnp.int32(np.int32(np.uint32(v & 0xFFFFFFFF).astype(np.int32)))


def _rotl(x, r):
    return lax.shift_left(x, _i32(r)) | lax.shift_right_logical(x, _i32(32 - r))


def _threefry_bits(x1):
    x0 = x1
    x1 = x0 ^ _rotl(x1, 13)
    for r in (15, 26, 6):
        x0 = x0 + x1
        x1 = x0 ^ _rotl(x1, r)
    x0 = x0 + _i32(KS1)
    x1 = x1 + _i32(KS2 + 1)
    for r in (17, 29, 16, 24):
        x0 = x0 + x1
        x1 = x0 ^ _rotl(x1, r)
    x0 = x0 + _i32(KS2)
    x1 = x1 + _i32(2)
    for r in (13, 15, 26, 6):
        x0 = x0 + x1
        x1 = x0 ^ _rotl(x1, r)
    x1 = x1 + _i32(KS1 + 3)
    for r in (17, 29, 16, 24):
        x0 = x0 + x1
        x1 = x0 ^ _rotl(x1, r)
    x0 = x0 + _i32(KS1)
    x1 = x1 + _i32(KS2 + 4)
    for r in (13, 15, 26, 6):
        x0 = x0 + x1
        x1 = x0 ^ _rotl(x1, r)
    x0 = x0 + _i32(KS2)
    x1 = x1 + _i32(5)
    return x0 ^ x1


def _log_softmax_rows(a):
    m = jnp.max(a, axis=-1, keepdims=True)
    sh = a - m
    return sh - jnp.log(jnp.sum(jnp.exp(sh), axis=-1, keepdims=True))


def _tc_body(x_ref, rows_ref, p_ref, w2t_ref, cnt_ref,
             qz_ref, pz_ref, idx_ref, pkl_ref, lpt_ref, lpm_ref, prk_ref):
    g = pl.program_id(0)
    xb = x_ref[...]
    pz_row = rows_ref[0:1, :]
    mu_row = rows_ref[1:2, :]
    opt_row = rows_ref[2:3, :]
    b2_row = rows_ref[3:4, :]
    w1_row = p_ref[0:1, :]
    b1_row = p_ref[1:2, :]

    h = jnp.tanh(xb * w1_row + b1_row)
    qz_raw = lax.dot_general(
        h, w2t_ref[...], (((1,), (0,)), ((), ())),
        precision=lax.Precision.HIGHEST,
        preferred_element_type=jnp.float32) + b2_row
    qz = _log_softmax_rows(qz_raw)
    qz_ref[...] = qz

    pz_ref[...] = jnp.broadcast_to(pz_row, (R, C))

    lp = -0.5 * ((xb - mu_row) / 5.0) ** 2 + LP_CONST
    tos = jnp.sum(opt_row)
    tl = _log_softmax_rows(lp + tos)
    pkl_part = jnp.sum(jnp.exp(qz) * (qz - tl))

    opt_lsm = _log_softmax_rows(opt_row)
    z1 = opt_lsm + lp
    m1 = jnp.max(z1, axis=-1, keepdims=True)
    lpt_part = jnp.sum(m1) + jnp.sum(
        jnp.log(jnp.sum(jnp.exp(z1 - m1), axis=-1)))
    z2 = pz_row + lp
    m2 = jnp.max(z2, axis=-1, keepdims=True)
    lpm_part = jnp.sum(m2) + jnp.sum(
        jnp.log(jnp.sum(jnp.exp(z2 - m2), axis=-1)))

    invB = np.float32(1.0 / B)
    is_last = g == G - 1
    pkl_acc = jnp.where(g == 0, pkl_part, pkl_ref[0, 0] + pkl_part)
    pkl_ref[0, 0] = jnp.where(is_last, pkl_acc * invB, pkl_acc)
    lpt_acc = jnp.where(g == 0, lpt_part, lpt_ref[0, 0] + lpt_part)
    lpt_ref[0, 0] = jnp.where(is_last, lpt_acc * invB, lpt_acc)
    lpm_acc = jnp.where(g == 0, lpm_part, lpm_ref[0, 0] + lpm_part)
    lpm_ref[0, 0] = jnp.where(is_last, lpm_acc * invB, lpm_acc)

    @pl.when(g == 0)
    def _():
        prk_ref[0, 0] = jnp.sum(jnp.exp(pz_row) * (pz_row - opt_lsm))

    x1 = cnt_ref[...] + g * _i32(R * S * C)
    bits = _threefry_bits(x1)
    fb = lax.shift_right_logical(bits, _i32(9)) | _i32(EXP_ONE)
    u = lax.bitcast_convert_type(fb, jnp.float32) - np.float32(1.0)
    nw_row = rows_ref[4:5, :]
    nscore = jnp.log(u) * nw_row
    idx_ref[...] = jnp.argmin(nscore, axis=-1).astype(jnp.int32).reshape(R, S)


def _tc_call(x, rows8, p8, w2t, cnt0):
    scal = jax.ShapeDtypeStruct((1, 1), jnp.float32)
    smem = pl.BlockSpec(memory_space=pltpu.SMEM)
    return pl.pallas_call(
        _tc_body,
        grid=(G,),
        in_specs=[
            pl.BlockSpec((R, 1), lambda g: (g, 0)),
            pl.BlockSpec((8, C), lambda g: (0, 0)),
            pl.BlockSpec((8, HDIM), lambda g: (0, 0)),
            pl.BlockSpec((HDIM, C), lambda g: (0, 0)),
            pl.BlockSpec((R * S, C), lambda g: (0, 0)),
        ],
        out_specs=[
            pl.BlockSpec((R, C), lambda g: (g, 0)),
            pl.BlockSpec((R, C), lambda g: (g, 0)),
            pl.BlockSpec((R, S), lambda g: (g, 0)),
            smem, smem, smem, smem,
        ],
        out_shape=[
            jax.ShapeDtypeStruct((B, C), jnp.float32),
            jax.ShapeDtypeStruct((B, C), jnp.float32),
            jax.ShapeDtypeStruct((B, S), jnp.int32),
            scal, scal, scal, scal,
        ],
    )(x, rows8, p8, w2t, cnt0)


_NC = 2
_NS = 16
_NW = _NC * _NS
_PERW = BS // _NW


def _sc_gather_call(idx_flat, pz_flat, mu_flat):
    mesh = plsc.VectorSubcoreMesh(core_axis_name="c", subcore_axis_name="s")

    @functools.partial(
        pl.kernel,
        mesh=mesh,
        compiler_params=pltpu.CompilerParams(needs_layout_passes=False),
        out_type=[
            jax.ShapeDtypeStruct((BS,), jnp.float32),
            jax.ShapeDtypeStruct((BS,), jnp.float32),
        ],
        scratch_types=[
            pltpu.VMEM((_PERW,), jnp.int32),
            pltpu.VMEM((C,), jnp.float32),
            pltpu.VMEM((C,), jnp.float32),
            pltpu.VMEM((_PERW,), jnp.float32),
            pltpu.VMEM((_PERW,), jnp.float32),
        ],
    )
    def sc_kernel(idx_hbm, pz_hbm, mu_hbm, ls_hbm, loc_hbm,
                  idx_v, pz_v, mu_v, ls_v, loc_v):
        wid = lax.axis_index("s") * _NC + lax.axis_index("c")
        base = wid * _PERW
        pltpu.sync_copy(idx_hbm.at[pl.ds(base, _PERW)], idx_v)
        pltpu.sync_copy(pz_hbm, pz_v)
        pltpu.sync_copy(mu_hbm, mu_v)

        def body(i, carry):
            sl = pl.ds(i * 16, 16)
            iv = idx_v[sl]
            ls_v[sl] = plsc.load_gather(pz_v, [iv])
            loc_v[sl] = plsc.load_gather(mu_v, [iv])
            return carry

        lax.fori_loop(0, _PERW // 16, body, 0)
        pltpu.sync_copy(ls_v, ls_hbm.at[pl.ds(base, _PERW)])
        pltpu.sync_copy(loc_v, loc_hbm.at[pl.ds(base, _PERW)])

    return sc_kernel(idx_flat, pz_flat, mu_flat)


def kernel(x, n_samples, log_theta, w1, b1, w2, b2, log_theta_opt, p_mu):
    pz_row = jax.nn.log_softmax(log_theta, axis=-1)
    mu_row = p_mu.reshape(1, C)
    rows8 = jnp.concatenate(
        [pz_row, mu_row, log_theta_opt, b2.reshape(1, C),
         -jnp.exp(-pz_row),
         jnp.zeros((3, C), jnp.float32)], axis=0)
    p8 = jnp.concatenate(
        [w1.reshape(1, HDIM), b1.reshape(1, HDIM),
         jnp.zeros((6, HDIM), jnp.float32)], axis=0)
    w2t = w2.T
    cnt0 = (jnp.arange(R * S * C, dtype=jnp.int32).reshape(R * S, C)
            + np.int32(KS1))

    qz, pz, idx, pkl, lpt, lpm, prk = _tc_call(x, rows8, p8, w2t, cnt0)

    idx_flat = idx.reshape(BS)
    ls_flat, loc_flat = _sc_gather_call(idx_flat, pz_row.reshape(C),
                                        p_mu.reshape(C))

    log_s = ls_flat.reshape(B, S)
    px_loc = loc_flat.reshape(B, S, 1)
    px_scale = jnp.full((B, S, 1), np.float32(5.0), jnp.float32)
    return (px_loc, px_scale, idx, log_s, qz, pz,
            prk.reshape(()), pkl.reshape(()),
            lpt.reshape(()), lpm.reshape(()))


# speedup vs baseline: 1.0361x; 1.0003x over previous
import functools

import numpy as np
import jax
import jax.numpy as jnp
from jax import lax
from jax.experimental import pallas as pl
from jax.experimental.pallas import tpu as pltpu
from jax.experimental.pallas import tpu_sc as plsc

B = 8192
C = 1024
HDIM = 16
S = 8
R = 128
G = B // R
BS = B * S

EXP_ONE = 0x3F800000
LP_CONST = np.float32(-np.log(5.0) - 0.5 * np.log(2.0 * np.pi))

KS1 = 42
KS2 = 42 ^ 0x1BD11BDA


def _i32(v):
    return jnp.int32(np.int32(np.uint32(v & 0xFFFFFFFF).astype(np.int32)))


def _rotl(x, r):
    return lax.shift_left(x, _i32(r)) | lax.shift_right_logical(x, _i32(32 - r))


def _threefry_bits(x1):
    x0 = x1
    x1 = x0 ^ _rotl(x1, 13)
    for r in (15, 26, 6):
        x0 = x0 + x1
        x1 = x0 ^ _rotl(x1, r)
    x0 = x0 + _i32(KS1)
    x1 = x1 + _i32(KS2 + 1)
    for r in (17, 29, 16, 24):
        x0 = x0 + x1
        x1 = x0 ^ _rotl(x1, r)
    x0 = x0 + _i32(KS2)
    x1 = x1 + _i32(2)
    for r in (13, 15, 26, 6):
        x0 = x0 + x1
        x1 = x0 ^ _rotl(x1, r)
    x1 = x1 + _i32(KS1 + 3)
    for r in (17, 29, 16, 24):
        x0 = x0 + x1
        x1 = x0 ^ _rotl(x1, r)
    x0 = x0 + _i32(KS1)
    x1 = x1 + _i32(KS2 + 4)
    for r in (13, 15, 26, 6):
        x0 = x0 + x1
        x1 = x0 ^ _rotl(x1, r)
    x0 = x0 + _i32(KS2)
    x1 = x1 + _i32(5)
    return x0 ^ x1


def _log_softmax_rows(a):
    m = jnp.max(a, axis=-1, keepdims=True)
    sh = a - m
    return sh - jnp.log(jnp.sum(jnp.exp(sh), axis=-1, keepdims=True))


def _tc_body(x_ref, rows_ref, p_ref, w2t_ref, cnt_ref,
             qz_ref, pz_ref, idx_ref, pkl_ref, lpt_ref, lpm_ref, prk_ref):
    g = pl.program_id(0)
    xb = x_ref[...]
    pz_row = rows_ref[0:1, :]
    mu_row = rows_ref[1:2, :]
    opt_row = rows_ref[2:3, :]
    b2_row = rows_ref[3:4, :]
    w1_row = p_ref[0:1, :]
    b1_row = p_ref[1:2, :]

    h = jnp.tanh(xb * w1_row + b1_row)
    qz_raw = lax.dot_general(
        h, w2t_ref[...], (((1,), (0,)), ((), ())),
        precision=lax.Precision.HIGHEST,
        preferred_element_type=jnp.float32) + b2_row
    qz = _log_softmax_rows(qz_raw)
    qz_ref[...] = qz

    pz_ref[...] = jnp.broadcast_to(pz_row, (R, C))

    lp = -0.5 * ((xb - mu_row) / 5.0) ** 2 + LP_CONST
    tos = jnp.sum(opt_row)
    tl = _log_softmax_rows(lp + tos)
    pkl_part = jnp.sum(jnp.exp(qz) * (qz - tl))

    opt_lsm = _log_softmax_rows(opt_row)
    z1 = opt_lsm + lp
    m1 = jnp.max(z1, axis=-1, keepdims=True)
    lpt_part = jnp.sum(m1) + jnp.sum(
        jnp.log(jnp.sum(jnp.exp(z1 - m1), axis=-1)))
    z2 = pz_row + lp
    m2 = jnp.max(z2, axis=-1, keepdims=True)
    lpm_part = jnp.sum(m2) + jnp.sum(
        jnp.log(jnp.sum(jnp.exp(z2 - m2), axis=-1)))

    invB = np.float32(1.0 / B)
    is_last = g == G - 1
    pkl_acc = jnp.where(g == 0, pkl_part, pkl_ref[0, 0] + pkl_part)
    pkl_ref[0, 0] = jnp.where(is_last, pkl_acc * invB, pkl_acc)
    lpt_acc = jnp.where(g == 0, lpt_part, lpt_ref[0, 0] + lpt_part)
    lpt_ref[0, 0] = jnp.where(is_last, lpt_acc * invB, lpt_acc)
    lpm_acc = jnp.where(g == 0, lpm_part, lpm_ref[0, 0] + lpm_part)
    lpm_ref[0, 0] = jnp.where(is_last, lpm_acc * invB, lpm_acc)

    @pl.when(g == 0)
    def _():
        prk_ref[0, 0] = jnp.sum(jnp.exp(pz_row) * (pz_row - opt_lsm))

    x1 = cnt_ref[...] + g * _i32(R * S * C)
    bits = _threefry_bits(x1)
    fb = lax.shift_right_logical(bits, _i32(9)) | _i32(EXP_ONE)
    u = lax.bitcast_convert_type(fb, jnp.float32) - np.float32(1.0)
    nw_row = rows_ref[4:5, :]
    nscore = jnp.log(u) * nw_row
    idx_ref[...] = jnp.argmin(nscore, axis=-1).astype(jnp.int32).reshape(R, S)


def _tc_call(x, rows8, p8, w2t, cnt0):
    scal = jax.ShapeDtypeStruct((1, 1), jnp.float32)
    smem = pl.BlockSpec(memory_space=pltpu.SMEM)
    return pl.pallas_call(
        _tc_body,
        grid=(G,),
        in_specs=[
            pl.BlockSpec((R, 1), lambda g: (g, 0)),
            pl.BlockSpec((8, C), lambda g: (0, 0)),
            pl.BlockSpec((8, HDIM), lambda g: (0, 0)),
            pl.BlockSpec((HDIM, C), lambda g: (0, 0)),
            pl.BlockSpec((R * S, C), lambda g: (0, 0)),
        ],
        out_specs=[
            pl.BlockSpec((R, C), lambda g: (g, 0)),
            pl.BlockSpec((R, C), lambda g: (g, 0)),
            pl.BlockSpec((R, S), lambda g: (g, 0)),
            smem, smem, smem, smem,
        ],
        out_shape=[
            jax.ShapeDtypeStruct((B, C), jnp.float32),
            jax.ShapeDtypeStruct((B, C), jnp.float32),
            jax.ShapeDtypeStruct((B, S), jnp.int32),
            scal, scal, scal, scal,
        ],
    )(x, rows8, p8, w2t, cnt0)


_NC = 2
_NS = 16
_NW = _NC * _NS
_PERW = BS // _NW


def _sc_gather_call(idx_flat, pz_flat, mu_flat):
    mesh = plsc.VectorSubcoreMesh(core_axis_name="c", subcore_axis_name="s")

    @functools.partial(
        pl.kernel,
        mesh=mesh,
        compiler_params=pltpu.CompilerParams(needs_layout_passes=False),
        out_type=[
            jax.ShapeDtypeStruct((BS,), jnp.float32),
            jax.ShapeDtypeStruct((BS,), jnp.float32),
        ],
        scratch_types=[
            pltpu.VMEM((_PERW,), jnp.int32),
            pltpu.VMEM((C,), jnp.float32),
            pltpu.VMEM((C,), jnp.float32),
            pltpu.VMEM((_PERW,), jnp.float32),
            pltpu.VMEM((_PERW,), jnp.float32),
        ],
    )
    def sc_kernel(idx_hbm, pz_hbm, mu_hbm, ls_hbm, loc_hbm,
                  idx_v, pz_v, mu_v, ls_v, loc_v):
        wid = lax.axis_index("s") * _NC + lax.axis_index("c")
        base = wid * _PERW
        pltpu.sync_copy(idx_hbm.at[pl.ds(base, _PERW)], idx_v)
        pltpu.sync_copy(pz_hbm, pz_v)
        pltpu.sync_copy(mu_hbm, mu_v)

        def body(i, carry):
            sl = pl.ds(i * 16, 16)
            iv = idx_v[sl]
            ls_v[sl] = plsc.load_gather(pz_v, [iv])
            loc_v[sl] = plsc.load_gather(mu_v, [iv])
            return carry

        lax.fori_loop(0, _PERW // 16, body, 0)
        pltpu.sync_copy(ls_v, ls_hbm.at[pl.ds(base, _PERW)])
        pltpu.sync_copy(loc_v, loc_hbm.at[pl.ds(base, _PERW)])

    return sc_kernel(idx_flat, pz_flat, mu_flat)


def kernel(x, n_samples, log_theta, w1, b1, w2, b2, log_theta_opt, p_mu):
    pz_row = jax.nn.log_softmax(log_theta, axis=-1)
    mu_row = p_mu.reshape(1, C)
    rows8 = jnp.concatenate(
        [pz_row, mu_row, log_theta_opt, b2.reshape(1, C),
         -jnp.exp(-pz_row),
         jnp.zeros((3, C), jnp.float32)], axis=0)
    p8 = jnp.concatenate(
        [w1.reshape(1, HDIM), b1.reshape(1, HDIM),
         jnp.zeros((6, HDIM), jnp.float32)], axis=0)
    w2t = w2.T
    cnt0 = (jnp.arange(R * S * C, dtype=jnp.int32).reshape(R * S, C)
            + np.int32(KS1))

    qz, pz, idx, pkl, lpt, lpm, prk = _tc_call(x, rows8, p8, w2t, cnt0)

    idx_flat = idx.reshape(BS)
    ls_flat, loc_flat = _sc_gather_call(idx_flat, pz_row.reshape(C),
                                        p_mu.reshape(C))

    log_s = ls_flat.reshape(B, S)
    px_loc = loc_flat.reshape(B, S, 1)
    px_scale = jnp.full((B, S, 1), np.float32(5.0), jnp.float32)
    return (px_loc, px_scale, idx, log_s, qz, pz,
            prk.reshape(()), pkl.reshape(()),
            lpt.reshape(()), lpm.reshape(()))


# speedup vs baseline: 1.0366x; 1.0004x over previous
import functools

import numpy as np
import jax
import jax.numpy as jnp
from jax import lax
from jax.experimental import pallas as pl
from jax.experimental.pallas import tpu as pltpu
from jax.experimental.pallas import tpu_sc as plsc

B = 8192
C = 1024
HDIM = 16
S = 8
R = 128
G = B // R
BS = B * S

EXP_ONE = 0x3F800000
LP_CONST = np.float32(-np.log(5.0) - 0.5 * np.log(2.0 * np.pi))

KS1 = 42
KS2 = 42 ^ 0x1BD11BDA


def _i32(v):
    return jnp.int32(np.int32(np.uint32(v & 0xFFFFFFFF).astype(np.int32)))


def _rotl(x, r):
    return lax.shift_left(x, _i32(r)) | lax.shift_right_logical(x, _i32(32 - r))


def _threefry_bits(x1):
    x0 = x1
    x1 = x0 ^ _rotl(x1, 13)
    for r in (15, 26, 6):
        x0 = x0 + x1
        x1 = x0 ^ _rotl(x1, r)
    x0 = x0 + _i32(KS1)
    x1 = x1 + _i32(KS2 + 1)
    for r in (17, 29, 16, 24):
        x0 = x0 + x1
        x1 = x0 ^ _rotl(x1, r)
    x0 = x0 + _i32(KS2)
    x1 = x1 + _i32(2)
    for r in (13, 15, 26, 6):
        x0 = x0 + x1
        x1 = x0 ^ _rotl(x1, r)
    x1 = x1 + _i32(KS1 + 3)
    for r in (17, 29, 16, 24):
        x0 = x0 + x1
        x1 = x0 ^ _rotl(x1, r)
    x0 = x0 + _i32(KS1)
    x1 = x1 + _i32(KS2 + 4)
    for r in (13, 15, 26, 6):
        x0 = x0 + x1
        x1 = x0 ^ _rotl(x1, r)
    x0 = x0 + _i32(KS2)
    x1 = x1 + _i32(5)
    return x0 ^ x1


def _log_softmax_rows(a):
    m = jnp.max(a, axis=-1, keepdims=True)
    sh = a - m
    return sh - jnp.log(jnp.sum(jnp.exp(sh), axis=-1, keepdims=True))


def _tc_body(x_ref, rows_ref, p_ref, w2t_ref, cnt_ref,
             qz_ref, pz_ref, idx_ref, pkl_ref, lpt_ref, lpm_ref, prk_ref):
    g = pl.program_id(0)
    xb = x_ref[...]
    pz_row = rows_ref[0:1, :]
    mu_row = rows_ref[1:2, :]
    opt_row = rows_ref[2:3, :]
    b2_row = rows_ref[3:4, :]
    w1_row = p_ref[0:1, :]
    b1_row = p_ref[1:2, :]

    h = jnp.tanh(xb * w1_row + b1_row)
    qz_raw = lax.dot_general(
        h, w2t_ref[...], (((1,), (0,)), ((), ())),
        precision=lax.Precision.HIGHEST,
        preferred_element_type=jnp.float32) + b2_row
    qz = _log_softmax_rows(qz_raw)
    qz_ref[...] = qz

    pz_ref[...] = jnp.broadcast_to(pz_row, (R, C))

    lp = -0.5 * ((xb - mu_row) / 5.0) ** 2 + LP_CONST
    tos = jnp.sum(opt_row)
    tl = _log_softmax_rows(lp + tos)
    pkl_part = jnp.sum(jnp.exp(qz) * (qz - tl))

    opt_lsm = _log_softmax_rows(opt_row)
    z1 = opt_lsm + lp
    m1 = jnp.max(z1, axis=-1, keepdims=True)
    lpt_part = jnp.sum(m1) + jnp.sum(
        jnp.log(jnp.sum(jnp.exp(z1 - m1), axis=-1)))
    z2 = pz_row + lp
    m2 = jnp.max(z2, axis=-1, keepdims=True)
    lpm_part = jnp.sum(m2) + jnp.sum(
        jnp.log(jnp.sum(jnp.exp(z2 - m2), axis=-1)))

    invB = np.float32(1.0 / B)
    is_last = g == G - 1
    pkl_acc = jnp.where(g == 0, pkl_part, pkl_ref[0, 0] + pkl_part)
    pkl_ref[0, 0] = jnp.where(is_last, pkl_acc * invB, pkl_acc)
    lpt_acc = jnp.where(g == 0, lpt_part, lpt_ref[0, 0] + lpt_part)
    lpt_ref[0, 0] = jnp.where(is_last, lpt_acc * invB, lpt_acc)
    lpm_acc = jnp.where(g == 0, lpm_part, lpm_ref[0, 0] + lpm_part)
    lpm_ref[0, 0] = jnp.where(is_last, lpm_acc * invB, lpm_acc)

    @pl.when(g == 0)
    def _():
        prk_ref[0, 0] = jnp.sum(jnp.exp(pz_row) * (pz_row - opt_lsm))

    NCHUNK = 2
    CR = (R * S) // NCHUNK
    nw_row = rows_ref[4:5, :]
    for h in range(NCHUNK):
        x1 = cnt_ref[h * CR:(h + 1) * CR, :] + g * _i32(R * S * C)
        bits = _threefry_bits(x1)
        fb = lax.shift_right_logical(bits, _i32(9)) | _i32(EXP_ONE)
        u = lax.bitcast_convert_type(fb, jnp.float32) - np.float32(1.0)
        nscore = jnp.log(u) * nw_row
        idx_ref[h * (R // NCHUNK):(h + 1) * (R // NCHUNK), :] = (
            jnp.argmin(nscore, axis=-1).astype(jnp.int32)
            .reshape(R // NCHUNK, S))


def _tc_call(x, rows8, p8, w2t, cnt0):
    scal = jax.ShapeDtypeStruct((1, 1), jnp.float32)
    smem = pl.BlockSpec(memory_space=pltpu.SMEM)
    return pl.pallas_call(
        _tc_body,
        grid=(G,),
        in_specs=[
            pl.BlockSpec((R, 1), lambda g: (g, 0)),
            pl.BlockSpec((8, C), lambda g: (0, 0)),
            pl.BlockSpec((8, HDIM), lambda g: (0, 0)),
            pl.BlockSpec((HDIM, C), lambda g: (0, 0)),
            pl.BlockSpec((R * S, C), lambda g: (0, 0)),
        ],
        out_specs=[
            pl.BlockSpec((R, C), lambda g: (g, 0)),
            pl.BlockSpec((R, C), lambda g: (g, 0)),
            pl.BlockSpec((R, S), lambda g: (g, 0)),
            smem, smem, smem, smem,
        ],
        out_shape=[
            jax.ShapeDtypeStruct((B, C), jnp.float32),
            jax.ShapeDtypeStruct((B, C), jnp.float32),
            jax.ShapeDtypeStruct((B, S), jnp.int32),
            scal, scal, scal, scal,
        ],
    )(x, rows8, p8, w2t, cnt0)


_NC = 2
_NS = 16
_NW = _NC * _NS
_PERW = BS // _NW


def _sc_gather_call(idx_flat, pz_flat, mu_flat):
    mesh = plsc.VectorSubcoreMesh(core_axis_name="c", subcore_axis_name="s")

    @functools.partial(
        pl.kernel,
        mesh=mesh,
        compiler_params=pltpu.CompilerParams(needs_layout_passes=False),
        out_type=[
            jax.ShapeDtypeStruct((BS,), jnp.float32),
            jax.ShapeDtypeStruct((BS,), jnp.float32),
        ],
        scratch_types=[
            pltpu.VMEM((_PERW,), jnp.int32),
            pltpu.VMEM((C,), jnp.float32),
            pltpu.VMEM((C,), jnp.float32),
            pltpu.VMEM((_PERW,), jnp.float32),
            pltpu.VMEM((_PERW,), jnp.float32),
        ],
    )
    def sc_kernel(idx_hbm, pz_hbm, mu_hbm, ls_hbm, loc_hbm,
                  idx_v, pz_v, mu_v, ls_v, loc_v):
        wid = lax.axis_index("s") * _NC + lax.axis_index("c")
        base = wid * _PERW
        pltpu.sync_copy(idx_hbm.at[pl.ds(base, _PERW)], idx_v)
        pltpu.sync_copy(pz_hbm, pz_v)
        pltpu.sync_copy(mu_hbm, mu_v)

        def body(i, carry):
            sl = pl.ds(i * 16, 16)
            iv = idx_v[sl]
            ls_v[sl] = plsc.load_gather(pz_v, [iv])
            loc_v[sl] = plsc.load_gather(mu_v, [iv])
            return carry

        lax.fori_loop(0, _PERW // 16, body, 0)
        pltpu.sync_copy(ls_v, ls_hbm.at[pl.ds(base, _PERW)])
        pltpu.sync_copy(loc_v, loc_hbm.at[pl.ds(base, _PERW)])

    return sc_kernel(idx_flat, pz_flat, mu_flat)


def kernel(x, n_samples, log_theta, w1, b1, w2, b2, log_theta_opt, p_mu):
    pz_row = jax.nn.log_softmax(log_theta, axis=-1)
    mu_row = p_mu.reshape(1, C)
    rows8 = jnp.concatenate(
        [pz_row, mu_row, log_theta_opt, b2.reshape(1, C),
         -jnp.exp(-pz_row),
         jnp.zeros((3, C), jnp.float32)], axis=0)
    p8 = jnp.concatenate(
        [w1.reshape(1, HDIM), b1.reshape(1, HDIM),
         jnp.zeros((6, HDIM), jnp.float32)], axis=0)
    w2t = w2.T
    cnt0 = (jnp.arange(R * S * C, dtype=jnp.int32).reshape(R * S, C)
            + np.int32(KS1))

    qz, pz, idx, pkl, lpt, lpm, prk = _tc_call(x, rows8, p8, w2t, cnt0)

    idx_flat = idx.reshape(BS)
    ls_flat, loc_flat = _sc_gather_call(idx_flat, pz_row.reshape(C),
                                        p_mu.reshape(C))

    log_s = ls_flat.reshape(B, S)
    px_loc = loc_flat.reshape(B, S, 1)
    px_scale = jnp.full((B, S, 1), np.float32(5.0), jnp.float32)
    return (px_loc, px_scale, idx, log_s, qz, pz,
            prk.reshape(()), pkl.reshape(()),
            lpt.reshape(()), lpm.reshape(()))


# speedup vs baseline: 1.0368x; 1.0003x over previous
import functools

import numpy as np
import jax
import jax.numpy as jnp
from jax import lax
from jax.experimental import pallas as pl
from jax.experimental.pallas import tpu as pltpu
from jax.experimental.pallas import tpu_sc as plsc

B = 8192
C = 1024
HDIM = 16
S = 8
R = 128
G = B // R
BS = B * S

EXP_ONE = 0x3F800000
LP_CONST = np.float32(-np.log(5.0) - 0.5 * np.log(2.0 * np.pi))

KS1 = 42
KS2 = 42 ^ 0x1BD11BDA


def _i32(v):
    return jnp.int32(np.int32(np.uint32(v & 0xFFFFFFFF).astype(np.int32)))


def _rotl(x, r):
    return lax.shift_left(x, _i32(r)) | lax.shift_right_logical(x, _i32(32 - r))


def _threefry_bits(x1):
    x0 = x1
    x1 = x0 ^ _rotl(x1, 13)
    for r in (15, 26, 6):
        x0 = x0 + x1
        x1 = x0 ^ _rotl(x1, r)
    x0 = x0 + _i32(KS1)
    x1 = x1 + _i32(KS2 + 1)
    for r in (17, 29, 16, 24):
        x0 = x0 + x1
        x1 = x0 ^ _rotl(x1, r)
    x0 = x0 + _i32(KS2)
    x1 = x1 + _i32(2)
    for r in (13, 15, 26, 6):
        x0 = x0 + x1
        x1 = x0 ^ _rotl(x1, r)
    x1 = x1 + _i32(KS1 + 3)
    for r in (17, 29, 16, 24):
        x0 = x0 + x1
        x1 = x0 ^ _rotl(x1, r)
    x0 = x0 + _i32(KS1)
    x1 = x1 + _i32(KS2 + 4)
    for r in (13, 15, 26, 6):
        x0 = x0 + x1
        x1 = x0 ^ _rotl(x1, r)
    x0 = x0 + _i32(KS2)
    x1 = x1 + _i32(5)
    return x0 ^ x1


def _log_softmax_rows(a):
    m = jnp.max(a, axis=-1, keepdims=True)
    sh = a - m
    return sh - jnp.log(jnp.sum(jnp.exp(sh), axis=-1, keepdims=True))


def _tc_body(x_ref, rows_ref, p_ref, w2t_ref, cnt_ref,
             qz_ref, pz_ref, idx_ref, pkl_ref, lpt_ref, lpm_ref, prk_ref):
    g = pl.program_id(0)
    xb = x_ref[...]
    pz_row = rows_ref[0:1, :]
    mu_row = rows_ref[1:2, :]
    opt_row = rows_ref[2:3, :]
    b2_row = rows_ref[3:4, :]
    w1_row = p_ref[0:1, :]
    b1_row = p_ref[1:2, :]

    h = jnp.tanh(xb * w1_row + b1_row)
    qz_raw = lax.dot_general(
        h, w2t_ref[...], (((1,), (0,)), ((), ())),
        precision=lax.Precision.HIGHEST,
        preferred_element_type=jnp.float32) + b2_row
    qz = _log_softmax_rows(qz_raw)
    qz_ref[...] = qz

    pz_ref[...] = jnp.broadcast_to(pz_row, (R, C))

    lp = -0.5 * ((xb - mu_row) / 5.0) ** 2 + LP_CONST
    tos = jnp.sum(opt_row)
    tl = _log_softmax_rows(lp + tos)
    pkl_part = jnp.sum(jnp.exp(qz) * (qz - tl))

    opt_lsm = _log_softmax_rows(opt_row)
    z1 = opt_lsm + lp
    m1 = jnp.max(z1, axis=-1, keepdims=True)
    lpt_part = jnp.sum(m1) + jnp.sum(
        jnp.log(jnp.sum(jnp.exp(z1 - m1), axis=-1)))
    z2 = pz_row + lp
    m2 = jnp.max(z2, axis=-1, keepdims=True)
    lpm_part = jnp.sum(m2) + jnp.sum(
        jnp.log(jnp.sum(jnp.exp(z2 - m2), axis=-1)))

    invB = np.float32(1.0 / B)
    is_last = g == G - 1
    pkl_acc = jnp.where(g == 0, pkl_part, pkl_ref[0, 0] + pkl_part)
    pkl_ref[0, 0] = jnp.where(is_last, pkl_acc * invB, pkl_acc)
    lpt_acc = jnp.where(g == 0, lpt_part, lpt_ref[0, 0] + lpt_part)
    lpt_ref[0, 0] = jnp.where(is_last, lpt_acc * invB, lpt_acc)
    lpm_acc = jnp.where(g == 0, lpm_part, lpm_ref[0, 0] + lpm_part)
    lpm_ref[0, 0] = jnp.where(is_last, lpm_acc * invB, lpm_acc)

    @pl.when(g == 0)
    def _():
        prk_ref[0, 0] = jnp.sum(jnp.exp(pz_row) * (pz_row - opt_lsm))

    NCHUNK = 4
    CR = (R * S) // NCHUNK
    nw_row = rows_ref[4:5, :]
    for h in range(NCHUNK):
        x1 = cnt_ref[h * CR:(h + 1) * CR, :] + g * _i32(R * S * C)
        bits = _threefry_bits(x1)
        fb = lax.shift_right_logical(bits, _i32(9)) | _i32(EXP_ONE)
        u = lax.bitcast_convert_type(fb, jnp.float32) - np.float32(1.0)
        nscore = jnp.log(u) * nw_row
        idx_ref[h * (R // NCHUNK):(h + 1) * (R // NCHUNK), :] = (
            jnp.argmin(nscore, axis=-1).astype(jnp.int32)
            .reshape(R // NCHUNK, S))


def _tc_call(x, rows8, p8, w2t, cnt0):
    scal = jax.ShapeDtypeStruct((1, 1), jnp.float32)
    smem = pl.BlockSpec(memory_space=pltpu.SMEM)
    return pl.pallas_call(
        _tc_body,
        grid=(G,),
        in_specs=[
            pl.BlockSpec((R, 1), lambda g: (g, 0)),
            pl.BlockSpec((8, C), lambda g: (0, 0)),
            pl.BlockSpec((8, HDIM), lambda g: (0, 0)),
            pl.BlockSpec((HDIM, C), lambda g: (0, 0)),
            pl.BlockSpec((R * S, C), lambda g: (0, 0)),
        ],
        out_specs=[
            pl.BlockSpec((R, C), lambda g: (g, 0)),
            pl.BlockSpec((R, C), lambda g: (g, 0)),
            pl.BlockSpec((R, S), lambda g: (g, 0)),
            smem, smem, smem, smem,
        ],
        out_shape=[
            jax.ShapeDtypeStruct((B, C), jnp.float32),
            jax.ShapeDtypeStruct((B, C), jnp.float32),
            jax.ShapeDtypeStruct((B, S), jnp.int32),
            scal, scal, scal, scal,
        ],
    )(x, rows8, p8, w2t, cnt0)


_NC = 2
_NS = 16
_NW = _NC * _NS
_PERW = BS // _NW


def _sc_gather_call(idx_flat, pz_flat, mu_flat):
    mesh = plsc.VectorSubcoreMesh(core_axis_name="c", subcore_axis_name="s")

    @functools.partial(
        pl.kernel,
        mesh=mesh,
        compiler_params=pltpu.CompilerParams(needs_layout_passes=False),
        out_type=[
            jax.ShapeDtypeStruct((BS,), jnp.float32),
            jax.ShapeDtypeStruct((BS,), jnp.float32),
        ],
        scratch_types=[
            pltpu.VMEM((_PERW,), jnp.int32),
            pltpu.VMEM((C,), jnp.float32),
            pltpu.VMEM((C,), jnp.float32),
            pltpu.VMEM((_PERW,), jnp.float32),
            pltpu.VMEM((_PERW,), jnp.float32),
        ],
    )
    def sc_kernel(idx_hbm, pz_hbm, mu_hbm, ls_hbm, loc_hbm,
                  idx_v, pz_v, mu_v, ls_v, loc_v):
        wid = lax.axis_index("s") * _NC + lax.axis_index("c")
        base = wid * _PERW
        pltpu.sync_copy(idx_hbm.at[pl.ds(base, _PERW)], idx_v)
        pltpu.sync_copy(pz_hbm, pz_v)
        pltpu.sync_copy(mu_hbm, mu_v)

        def body(i, carry):
            sl = pl.ds(i * 16, 16)
            iv = idx_v[sl]
            ls_v[sl] = plsc.load_gather(pz_v, [iv])
            loc_v[sl] = plsc.load_gather(mu_v, [iv])
            return carry

        lax.fori_loop(0, _PERW // 16, body, 0)
        pltpu.sync_copy(ls_v, ls_hbm.at[pl.ds(base, _PERW)])
        pltpu.sync_copy(loc_v, loc_hbm.at[pl.ds(base, _PERW)])

    return sc_kernel(idx_flat, pz_flat, mu_flat)


def kernel(x, n_samples, log_theta, w1, b1, w2, b2, log_theta_opt, p_mu):
    pz_row = jax.nn.log_softmax(log_theta, axis=-1)
    mu_row = p_mu.reshape(1, C)
    rows8 = jnp.concatenate(
        [pz_row, mu_row, log_theta_opt, b2.reshape(1, C),
         -jnp.exp(-pz_row),
         jnp.zeros((3, C), jnp.float32)], axis=0)
    p8 = jnp.concatenate(
        [w1.reshape(1, HDIM), b1.reshape(1, HDIM),
         jnp.zeros((6, HDIM), jnp.float32)], axis=0)
    w2t = w2.T
    cnt0 = (jnp.arange(R * S * C, dtype=jnp.int32).reshape(R * S, C)
            + np.int32(KS1))

    qz, pz, idx, pkl, lpt, lpm, prk = _tc_call(x, rows8, p8, w2t, cnt0)

    idx_flat = idx.reshape(BS)
    ls_flat, loc_flat = _sc_gather_call(idx_flat, pz_row.reshape(C),
                                        p_mu.reshape(C))

    log_s = ls_flat.reshape(B, S)
    px_loc = loc_flat.reshape(B, S, 1)
    px_scale = jnp.full((B, S, 1), np.float32(5.0), jnp.float32)
    return (px_loc, px_scale, idx, log_s, qz, pz,
            prk.reshape(()), pkl.reshape(()),
            lpt.reshape(()), lpm.reshape(()))
